# baseline (speedup 1.0000x reference)
import functools

import jax
import jax.numpy as jnp
from jax.experimental import pallas as pl
from jax.experimental.pallas import tpu as pltpu

B, S, H = 2, 4096, 1024
E, K = 64, 2
NB = 8
INTER = 256
N_SHARED = 2
SH_INTER = INTER * N_SHARED
EPS = 1e-8

TM = 512


def _moe_body(x_ref, xb_ref, wr_ref, wg_ref, wu_ref, wd_ref, amp_ref,
              scale_ref, wgsh_ref, wush_ref, wdsh_ref, o_ref):
    x = x_ref[...]
    xb = xb_ref[...]

    logits = jnp.dot(x, wr_ref[...], preferred_element_type=jnp.float32)
    e_iota = jax.lax.broadcasted_iota(jnp.int32, logits.shape, 1)
    v1 = jnp.max(logits, axis=-1, keepdims=True)
    i1 = jnp.min(jnp.where(logits == v1, e_iota, E), axis=-1, keepdims=True)
    mask1 = e_iota == i1
    rest = jnp.where(mask1, -jnp.inf, logits)
    v2 = jnp.max(rest, axis=-1, keepdims=True)
    i2 = jnp.min(jnp.where(rest == v2, e_iota, E), axis=-1, keepdims=True)
    mask2 = e_iota == i2
    w2 = jax.nn.sigmoid(v2 - v1)
    g = jnp.where(mask1, 1.0 - w2, jnp.where(mask2, w2, 0.0))

    a0 = amp_ref[0]
    a1 = amp_ref[1]
    ap = a0 * a0 + a1 * a1
    ap = ap / (jnp.sum(ap, axis=-1, keepdims=True) + EPS)
    amp_scaled = ap * scale_ref[...]

    c = jnp.dot(g, amp_scaled, preferred_element_type=jnp.float32)

    gate = jnp.dot(xb, wg_ref[...], preferred_element_type=jnp.float32)
    up = jnp.dot(xb, wu_ref[...], preferred_element_type=jnp.float32)
    basis = (gate * jax.nn.sigmoid(gate)) * up

    combined = c[:, 0:1] * basis[:, 0:INTER]
    for b in range(1, NB):
        combined = combined + c[:, b:b + 1] * basis[:, b * INTER:(b + 1) * INTER]

    sg = jnp.dot(xb, wgsh_ref[...], preferred_element_type=jnp.float32)
    su = jnp.dot(xb, wush_ref[...], preferred_element_type=jnp.float32)
    sh = (sg * jax.nn.sigmoid(sg)) * su

    o_ref[...] = (
        jnp.dot(combined.astype(jnp.bfloat16), wd_ref[...],
                preferred_element_type=jnp.float32)
        + jnp.dot(sh.astype(jnp.bfloat16), wdsh_ref[...],
                  preferred_element_type=jnp.float32)
    )


@jax.jit
def _moe_fused(x, W_router, W_gate, W_up, W_down, amp_t, scale_c,
               Wg_sh, Wu_sh, Wd_sh):
    T = x.shape[0]
    grid = (T // TM,)
    xb = x.astype(jnp.bfloat16)
    wg = W_gate.astype(jnp.bfloat16)
    wu = W_up.astype(jnp.bfloat16)
    wd = W_down.astype(jnp.bfloat16)
    wgsh = Wg_sh.astype(jnp.bfloat16)
    wush = Wu_sh.astype(jnp.bfloat16)
    wdsh = Wd_sh.astype(jnp.bfloat16)

    def tile(i):
        return (i, 0)

    def whole(i):
        return (0, 0)

    return pl.pallas_call(
        _moe_body,
        grid=grid,
        in_specs=[
            pl.BlockSpec((TM, H), tile),
            pl.BlockSpec((TM, H), tile),
            pl.BlockSpec((H, E), whole),
            pl.BlockSpec((H, NB * INTER), whole),
            pl.BlockSpec((H, NB * INTER), whole),
            pl.BlockSpec((INTER, H), whole),
            pl.BlockSpec((2, E, NB), lambda i: (0, 0, 0)),
            pl.BlockSpec((E, 1), whole),
            pl.BlockSpec((H, SH_INTER), whole),
            pl.BlockSpec((H, SH_INTER), whole),
            pl.BlockSpec((SH_INTER, H), whole),
        ],
        out_specs=pl.BlockSpec((TM, H), tile),
        out_shape=jax.ShapeDtypeStruct((T, H), jnp.float32),
    )(x, xb, W_router, wg, wu, wd, amp_t, scale_c, wgsh, wush, wdsh)


def kernel(hidden_states, W_router, W_gate, W_up, W_down, expert_amplitudes,
           expert_scale, Wg_sh, Wu_sh, Wd_sh):
    T = B * S
    x = hidden_states.reshape(T, H)
    amp_t = expert_amplitudes.transpose(2, 0, 1)
    scale_c = expert_scale.reshape(E, 1)
    out = _moe_fused(x, W_router, W_gate, W_up, W_down, amp_t, scale_c,
                     Wg_sh, Wu_sh, Wd_sh)
    return out.reshape(B, S, H)


# speedup vs baseline: 1.2073x; 1.2073x over previous
import functools

import jax
import jax.numpy as jnp
from jax.experimental import pallas as pl
from jax.experimental.pallas import tpu as pltpu

B, S, H = 2, 4096, 1024
E, K = 64, 2
NB = 8
INTER = 256
N_SHARED = 2
SH_INTER = INTER * N_SHARED
EPS = 1e-8

TM = 512


def _moe_body(x_ref, wr_ref, wg_ref, wu_ref, wd_ref, amp_ref,
              scale_ref, wgsh_ref, wush_ref, wdsh_ref, o_ref):
    x = x_ref[...]
    xb = x

    logits = jnp.dot(x, wr_ref[...], preferred_element_type=jnp.float32)
    e_iota = jax.lax.broadcasted_iota(jnp.int32, logits.shape, 1)
    v1 = jnp.max(logits, axis=-1, keepdims=True)
    i1 = jnp.min(jnp.where(logits == v1, e_iota, E), axis=-1, keepdims=True)
    mask1 = e_iota == i1
    rest = jnp.where(mask1, -jnp.inf, logits)
    v2 = jnp.max(rest, axis=-1, keepdims=True)
    i2 = jnp.min(jnp.where(rest == v2, e_iota, E), axis=-1, keepdims=True)
    mask2 = e_iota == i2
    w2 = jax.nn.sigmoid(v2 - v1)
    g = jnp.where(mask1, 1.0 - w2, jnp.where(mask2, w2, 0.0))

    a0 = amp_ref[0]
    a1 = amp_ref[1]
    ap = a0 * a0 + a1 * a1
    ap = ap / (jnp.sum(ap, axis=-1, keepdims=True) + EPS)
    amp_scaled = ap * scale_ref[...]

    c = jnp.dot(g, amp_scaled, preferred_element_type=jnp.float32)

    gate = jnp.dot(xb, wg_ref[...], preferred_element_type=jnp.float32)
    up = jnp.dot(xb, wu_ref[...], preferred_element_type=jnp.float32)
    basis = (gate * jax.nn.sigmoid(gate)) * up

    combined = c[:, 0:1] * basis[:, 0:INTER]
    for b in range(1, NB):
        combined = combined + c[:, b:b + 1] * basis[:, b * INTER:(b + 1) * INTER]

    sg = jnp.dot(xb, wgsh_ref[...], preferred_element_type=jnp.float32)
    su = jnp.dot(xb, wush_ref[...], preferred_element_type=jnp.float32)
    sh = (sg * jax.nn.sigmoid(sg)) * su

    o_ref[...] = (
        jnp.dot(combined, wd_ref[...], preferred_element_type=jnp.float32)
        + jnp.dot(sh, wdsh_ref[...], preferred_element_type=jnp.float32)
    )


@jax.jit
def _moe_fused(x, W_router, W_gate, W_up, W_down, amp_t, scale_c,
               Wg_sh, Wu_sh, Wd_sh):
    T = x.shape[0]
    grid = (T // TM,)
    wg, wu, wd, wgsh, wush, wdsh = W_gate, W_up, W_down, Wg_sh, Wu_sh, Wd_sh

    def tile(i):
        return (i, 0)

    def whole(i):
        return (0, 0)

    return pl.pallas_call(
        _moe_body,
        grid=grid,
        in_specs=[
            pl.BlockSpec((TM, H), tile),
            pl.BlockSpec((H, E), whole),
            pl.BlockSpec((H, NB * INTER), whole),
            pl.BlockSpec((H, NB * INTER), whole),
            pl.BlockSpec((INTER, H), whole),
            pl.BlockSpec((2, E, NB), lambda i: (0, 0, 0)),
            pl.BlockSpec((E, 1), whole),
            pl.BlockSpec((H, SH_INTER), whole),
            pl.BlockSpec((H, SH_INTER), whole),
            pl.BlockSpec((SH_INTER, H), whole),
        ],
        out_specs=pl.BlockSpec((TM, H), tile),
        out_shape=jax.ShapeDtypeStruct((T, H), jnp.float32),
    )(x, W_router, wg, wu, wd, amp_t, scale_c, wgsh, wush, wdsh)


def kernel(hidden_states, W_router, W_gate, W_up, W_down, expert_amplitudes,
           expert_scale, Wg_sh, Wu_sh, Wd_sh):
    T = B * S
    x = hidden_states.reshape(T, H)
    amp_t = expert_amplitudes.transpose(2, 0, 1)
    scale_c = expert_scale.reshape(E, 1)
    out = _moe_fused(x, W_router, W_gate, W_up, W_down, amp_t, scale_c,
                     Wg_sh, Wu_sh, Wd_sh)
    return out.reshape(B, S, H)


# speedup vs baseline: 1.2592x; 1.0430x over previous
import functools

import jax
import jax.numpy as jnp
from jax.experimental import pallas as pl
from jax.experimental.pallas import tpu as pltpu

B, S, H = 2, 4096, 1024
E, K = 64, 2
NB = 8
INTER = 256
N_SHARED = 2
SH_INTER = INTER * N_SHARED
EPS = 1e-8

TM = 1024


def _moe_body(x_ref, wr_ref, wg_ref, wu_ref, wd_ref, amp_ref,
              scale_ref, wgsh_ref, wush_ref, wdsh_ref, o_ref):
    x = x_ref[...]
    xb = x

    logits = jnp.dot(x, wr_ref[...], preferred_element_type=jnp.float32)
    e_iota = jax.lax.broadcasted_iota(jnp.int32, logits.shape, 1)
    v1 = jnp.max(logits, axis=-1, keepdims=True)
    i1 = jnp.min(jnp.where(logits == v1, e_iota, E), axis=-1, keepdims=True)
    mask1 = e_iota == i1
    rest = jnp.where(mask1, -jnp.inf, logits)
    v2 = jnp.max(rest, axis=-1, keepdims=True)
    i2 = jnp.min(jnp.where(rest == v2, e_iota, E), axis=-1, keepdims=True)
    mask2 = e_iota == i2
    w2 = jax.nn.sigmoid(v2 - v1)
    g = jnp.where(mask1, 1.0 - w2, jnp.where(mask2, w2, 0.0))

    a0 = amp_ref[0]
    a1 = amp_ref[1]
    ap = a0 * a0 + a1 * a1
    ap = ap / (jnp.sum(ap, axis=-1, keepdims=True) + EPS)
    amp_scaled = ap * scale_ref[...]

    c = jnp.dot(g, amp_scaled, preferred_element_type=jnp.float32)

    gate = jnp.dot(xb, wg_ref[...], preferred_element_type=jnp.float32)
    up = jnp.dot(xb, wu_ref[...], preferred_element_type=jnp.float32)
    basis = (gate * jax.nn.sigmoid(gate)) * up

    combined = c[:, 0:1] * basis[:, 0:INTER]
    for b in range(1, NB):
        combined = combined + c[:, b:b + 1] * basis[:, b * INTER:(b + 1) * INTER]

    sg = jnp.dot(xb, wgsh_ref[...], preferred_element_type=jnp.float32)
    su = jnp.dot(xb, wush_ref[...], preferred_element_type=jnp.float32)
    sh = (sg * jax.nn.sigmoid(sg)) * su

    o_ref[...] = (
        jnp.dot(combined, wd_ref[...], preferred_element_type=jnp.float32)
        + jnp.dot(sh, wdsh_ref[...], preferred_element_type=jnp.float32)
    )


@jax.jit
def _moe_fused(x, W_router, W_gate, W_up, W_down, amp_t, scale_c,
               Wg_sh, Wu_sh, Wd_sh):
    T = x.shape[0]
    grid = (T // TM,)
    wg, wu, wd, wgsh, wush, wdsh = W_gate, W_up, W_down, Wg_sh, Wu_sh, Wd_sh

    def tile(i):
        return (i, 0)

    def whole(i):
        return (0, 0)

    return pl.pallas_call(
        _moe_body,
        grid=grid,
        in_specs=[
            pl.BlockSpec((TM, H), tile),
            pl.BlockSpec((H, E), whole),
            pl.BlockSpec((H, NB * INTER), whole),
            pl.BlockSpec((H, NB * INTER), whole),
            pl.BlockSpec((INTER, H), whole),
            pl.BlockSpec((2, E, NB), lambda i: (0, 0, 0)),
            pl.BlockSpec((E, 1), whole),
            pl.BlockSpec((H, SH_INTER), whole),
            pl.BlockSpec((H, SH_INTER), whole),
            pl.BlockSpec((SH_INTER, H), whole),
        ],
        out_specs=pl.BlockSpec((TM, H), tile),
        out_shape=jax.ShapeDtypeStruct((T, H), jnp.float32),
    )(x, W_router, wg, wu, wd, amp_t, scale_c, wgsh, wush, wdsh)


def kernel(hidden_states, W_router, W_gate, W_up, W_down, expert_amplitudes,
           expert_scale, Wg_sh, Wu_sh, Wd_sh):
    T = B * S
    x = hidden_states.reshape(T, H)
    amp_t = expert_amplitudes.transpose(2, 0, 1)
    scale_c = expert_scale.reshape(E, 1)
    out = _moe_fused(x, W_router, W_gate, W_up, W_down, amp_t, scale_c,
                     Wg_sh, Wu_sh, Wd_sh)
    return out.reshape(B, S, H)
